# baseline (speedup 1.0000x reference)
import functools

import jax
import jax.numpy as jnp
from jax import lax
from jax.experimental import pallas as pl
from jax.experimental.pallas import tpu as pltpu
from jax.experimental.pallas import tpu_sc as plsc

N = 10000
NH = 3392
DF = 128
E = 320000
NL = 100000

NC = 2
NS = 16
APAD = 3456
DUMP = 3400
EPAD = 327680
ECH_T = 160
ECH_W = 80
NPAD = 10112
LPAD = 102400
LCH_W = 25

_MESH = plsc.VectorSubcoreMesh(
    core_axis_name="c", subcore_axis_name="s", num_cores=NC, num_subcores=NS)


def _fill16(ref, n, val):
  def body(i, _):
    ref[i, :] = jnp.full((16,), val, jnp.float32)
  lax.fori_loop(0, n, body, None)


def _zero_rows(zref, n):
  def body(i, _):
    for j in range(8):
      zref[i, pl.ds(j * 16, 16)] = jnp.zeros((16,), jnp.float32)
  lax.fori_loop(0, n, body, None)


@functools.partial(
    pl.kernel,
    out_type=jax.ShapeDtypeStruct((NC * NS, NPAD // 128, 128), jnp.float32),
    mesh=_MESH,
    scratch_types=[
        pltpu.VMEM((ECH_W, 128), jnp.int32),
        pltpu.VMEM((NPAD // 128, 128), jnp.float32),
    ],
)
def _deg_kernel(dst_hbm, degp_hbm, idx_v, hist_v):
  cid = lax.axis_index("c")
  sid = lax.axis_index("s")
  wid = sid * NC + cid
  _zero_rows(hist_v, NPAD // 128)
  pltpu.sync_copy(dst_hbm.at[pl.ds(wid * ECH_W, ECH_W)], idx_v)

  lanes = lax.iota(jnp.int32, 16)

  def upd(g, _):
    dvec = idx_v[g // 8, pl.ds((g % 8) * 16, 16)]
    for l in range(16):
      d = dvec[l]
      hr = d // 128
      hs = (d % 128) // 16 * 16
      hm = d % 16
      seg = hist_v[hr, pl.ds(hs, 16)]
      hist_v[hr, pl.ds(hs, 16)] = jnp.where(lanes == hm, seg + 1.0, seg)
  lax.fori_loop(0, ECH_W * 8, upd, None)
  pltpu.sync_copy(hist_v, degp_hbm.at[wid])


def _deg_sum(degp):
  def body(p_ref, o_ref):
    o_ref[...] = jnp.sum(p_ref[...], axis=0)
  return pl.pallas_call(
      body,
      grid=(1,),
      in_specs=[pl.BlockSpec((NC * NS, NPAD // 128, 128),
                             lambda i: (0, 0, 0))],
      out_specs=[pl.BlockSpec((NPAD // 128, 128), lambda i: (0, 0))],
      out_shape=[jax.ShapeDtypeStruct((NPAD // 128, 128), jnp.float32)],
  )(degp)[0]


@functools.partial(
    pl.kernel,
    out_type=[
        jax.ShapeDtypeStruct((APAD, DF), jnp.float32),
        jax.ShapeDtypeStruct((APAD, DF), jnp.float32),
        jax.ShapeDtypeStruct((APAD, DF), jnp.float32),
        jax.ShapeDtypeStruct((APAD, DF), jnp.float32),
        jax.ShapeDtypeStruct((APAD, DF), jnp.float32),
        jax.ShapeDtypeStruct((APAD, DF), jnp.float32),
    ],
    mesh=_MESH,
    scratch_types=[
        pltpu.VMEM((ECH_T, 128), jnp.int32),
        pltpu.VMEM((ECH_T, 128), jnp.int32),
        pltpu.VMEM((128, DF), jnp.float32),
        pltpu.VMEM((128, DF), jnp.float32),
        pltpu.VMEM((72, DF), jnp.float32),
        pltpu.VMEM_SHARED((APAD, DF), jnp.float32),
        pltpu.SemaphoreType.DMA,
        pltpu.SemaphoreType.DMA,
    ],
)
def _prop_kernel(v0_hbm, v1_hbm, src_hbm, dsta_hbm, dstb_hbm, dstc_hbm,
                 s0a_hbm, s0b_hbm, s0c_hbm, s1a_hbm, s1b_hbm, s1c_hbm,
                 srcv, dstv, rows_a, rows_b, zero_v, acc,
                 sem_a, sem_b):
  cid = lax.axis_index("c")
  sid = lax.axis_index("s")
  _zero_rows(zero_v, 72)

  pltpu.sync_copy(src_hbm.at[pl.ds(sid * ECH_T, ECH_T)], srcv)

  def one_pass(v_hbm, dst_hbm, s_hbm):
    pltpu.sync_copy(dst_hbm.at[pl.ds(sid * ECH_T, ECH_T)], dstv)
    for k in range(3):
      pltpu.sync_copy(zero_v, acc.at[pl.ds(sid * 216 + k * 72, 72)])
    plsc.subcore_barrier()

    pltpu.async_copy(v_hbm.at[srcv.at[0]], rows_a, sem_a)

    def step(g, _):
      ja = 2 * g
      jb = 2 * g + 1
      pltpu.make_async_copy(v_hbm.at[srcv.at[ja]], rows_a, sem_a).wait()
      pltpu.async_copy(v_hbm.at[srcv.at[jb]], rows_b, sem_b)
      pltpu.sync_copy(rows_a, acc.at[dstv.at[ja]], add=True)
      pltpu.make_async_copy(v_hbm.at[srcv.at[jb]], rows_b, sem_b).wait()

      @pl.when(ja + 2 < ECH_T)
      def _():
        pltpu.async_copy(v_hbm.at[srcv.at[ja + 2]], rows_a, sem_a)
      pltpu.sync_copy(rows_b, acc.at[dstv.at[jb]], add=True)
    lax.fori_loop(0, ECH_T // 2, step, None)
    plsc.subcore_barrier()
    pltpu.sync_copy(acc.at[pl.ds(sid * 216, 216)],
                    s_hbm.at[pl.ds(sid * 216, 216)])

  @pl.when(cid == 0)
  def _():
    one_pass(v0_hbm, dsta_hbm, s0a_hbm)
    one_pass(v0_hbm, dstb_hbm, s0b_hbm)
    one_pass(v0_hbm, dstc_hbm, s0c_hbm)

  @pl.when(cid == 1)
  def _():
    one_pass(v1_hbm, dsta_hbm, s1a_hbm)
    one_pass(v1_hbm, dstb_hbm, s1b_hbm)
    one_pass(v1_hbm, dstc_hbm, s1c_hbm)


@functools.partial(
    pl.kernel,
    out_type=[
        jax.ShapeDtypeStruct((LPAD, DF), jnp.float32),
        jax.ShapeDtypeStruct((LPAD, DF), jnp.float32),
    ],
    mesh=_MESH,
    scratch_types=[
        pltpu.VMEM((LCH_W * 128,), jnp.int32),
        pltpu.VMEM((LCH_W * 128,), jnp.int32),
        pltpu.VMEM((128, DF), jnp.float32),
        pltpu.VMEM((128, DF), jnp.float32),
        pltpu.SemaphoreType.DMA,
        pltpu.SemaphoreType.DMA,
    ],
)
def _decode_gather(z_hbm, ls_hbm, ld_hbm, zs_hbm, zd_hbm,
                   idxs, idxd, rows_a, rows_b, sem_a, sem_b):
  cid = lax.axis_index("c")
  sid = lax.axis_index("s")
  wid = sid * NC + cid
  nw = LCH_W * 128
  pltpu.sync_copy(ls_hbm.at[pl.ds(wid * nw, nw)], idxs)
  pltpu.sync_copy(ld_hbm.at[pl.ds(wid * nw, nw)], idxd)

  def step(j, _):
    isl = idxs.at[pl.ds(j * 128, 128)]
    idl = idxd.at[pl.ds(j * 128, 128)]
    base = wid * nw + j * 128
    pltpu.async_copy(z_hbm.at[isl], rows_a, sem_a)
    pltpu.async_copy(z_hbm.at[idl], rows_b, sem_b)
    pltpu.make_async_copy(z_hbm.at[isl], rows_a, sem_a).wait()
    pltpu.sync_copy(rows_a, zs_hbm.at[pl.ds(base, 128)])
    pltpu.make_async_copy(z_hbm.at[idl], rows_b, sem_b).wait()
    pltpu.sync_copy(rows_b, zd_hbm.at[pl.ds(base, 128)])
  lax.fori_loop(0, LCH_W, step, None)


_RB = 1000


def _dense1(x, w0, w1, degcol):
  def body(x_ref, w0_ref, w1_ref, dc_ref, v0_ref, v1_ref, di_ref):
    di = lax.rsqrt(dc_ref[...] + 1.0)
    xb = x_ref[...]
    v0_ref[...] = di * jnp.dot(xb, w0_ref[...], preferred_element_type=jnp.float32)
    v1_ref[...] = di * jnp.dot(xb, w1_ref[...], preferred_element_type=jnp.float32)
    di_ref[...] = di
  return pl.pallas_call(
      body,
      grid=(N // _RB,),
      in_specs=[
          pl.BlockSpec((_RB, DF), lambda i: (i, 0)),
          pl.BlockSpec((DF, DF), lambda i: (0, 0)),
          pl.BlockSpec((DF, DF), lambda i: (0, 0)),
          pl.BlockSpec((_RB, 1), lambda i: (i, 0)),
      ],
      out_specs=[
          pl.BlockSpec((_RB, DF), lambda i: (i, 0)),
          pl.BlockSpec((_RB, DF), lambda i: (i, 0)),
          pl.BlockSpec((_RB, 1), lambda i: (i, 0)),
      ],
      out_shape=[
          jax.ShapeDtypeStruct((N, DF), jnp.float32),
          jax.ShapeDtypeStruct((N, DF), jnp.float32),
          jax.ShapeDtypeStruct((N, 1), jnp.float32),
      ],
  )(x, w0, w1, degcol)


def _dense2(s0, s1, v0, v1, di, w20, w21, b10, b11):
  def body(s0_ref, s1_ref, v0_ref, v1_ref, di_ref, w20_ref, w21_ref,
           b10_ref, b11_ref, o0_ref, o1_ref):
    d = di_ref[...]
    h0 = d * (s0_ref[...] + v0_ref[...]) + b10_ref[...]
    h1 = d * (s1_ref[...] + v1_ref[...]) + b11_ref[...]
    o0_ref[...] = d * jnp.dot(h0, w20_ref[...], preferred_element_type=jnp.float32)
    o1_ref[...] = d * jnp.dot(h1, w21_ref[...], preferred_element_type=jnp.float32)
  return pl.pallas_call(
      body,
      grid=(N // _RB,),
      in_specs=[
          pl.BlockSpec((_RB, DF), lambda i: (i, 0)),
          pl.BlockSpec((_RB, DF), lambda i: (i, 0)),
          pl.BlockSpec((_RB, DF), lambda i: (i, 0)),
          pl.BlockSpec((_RB, DF), lambda i: (i, 0)),
          pl.BlockSpec((_RB, 1), lambda i: (i, 0)),
          pl.BlockSpec((DF, DF), lambda i: (0, 0)),
          pl.BlockSpec((DF, DF), lambda i: (0, 0)),
          pl.BlockSpec((1, DF), lambda i: (0, 0)),
          pl.BlockSpec((1, DF), lambda i: (0, 0)),
      ],
      out_specs=[
          pl.BlockSpec((_RB, DF), lambda i: (i, 0)),
          pl.BlockSpec((_RB, DF), lambda i: (i, 0)),
      ],
      out_shape=[
          jax.ShapeDtypeStruct((N, DF), jnp.float32),
          jax.ShapeDtypeStruct((N, DF), jnp.float32),
      ],
  )(s0, s1, v0, v1, di, w20, w21, b10, b11)


def _dense3(s0, s1, v0, v1, di, b20, b21, w30, w31, b30, b31,
            w40, w41, b40, b41):
  def body(s0_ref, s1_ref, v0_ref, v1_ref, di_ref, b20_ref, b21_ref,
           w30_ref, w31_ref, b30_ref, b31_ref, w40_ref, w41_ref,
           b40_ref, b41_ref, z_ref):
    d = di_ref[...]
    h0 = d * (s0_ref[...] + v0_ref[...]) + b20_ref[...]
    h1 = d * (s1_ref[...] + v1_ref[...]) + b21_ref[...]
    g0 = jnp.maximum(
        jnp.dot(h0, w30_ref[...], preferred_element_type=jnp.float32)
        + b30_ref[...], 0.0)
    g1 = jnp.maximum(
        jnp.dot(h1, w31_ref[...], preferred_element_type=jnp.float32)
        + b31_ref[...], 0.0)
    z0 = jnp.dot(g0, w40_ref[...], preferred_element_type=jnp.float32) + b40_ref[...]
    z1 = jnp.dot(g1, w41_ref[...], preferred_element_type=jnp.float32) + b41_ref[...]
    z_ref[...] = 0.5 * (z0 + z1)
  full = lambda i: (0, 0)
  return pl.pallas_call(
      body,
      grid=(N // _RB,),
      in_specs=[
          pl.BlockSpec((_RB, DF), lambda i: (i, 0)),
          pl.BlockSpec((_RB, DF), lambda i: (i, 0)),
          pl.BlockSpec((_RB, DF), lambda i: (i, 0)),
          pl.BlockSpec((_RB, DF), lambda i: (i, 0)),
          pl.BlockSpec((_RB, 1), lambda i: (i, 0)),
          pl.BlockSpec((1, DF), full),
          pl.BlockSpec((1, DF), full),
          pl.BlockSpec((DF, DF), full),
          pl.BlockSpec((DF, DF), full),
          pl.BlockSpec((1, DF), full),
          pl.BlockSpec((1, DF), full),
          pl.BlockSpec((DF, DF), full),
          pl.BlockSpec((DF, DF), full),
          pl.BlockSpec((1, DF), full),
          pl.BlockSpec((1, DF), full),
      ],
      out_specs=[pl.BlockSpec((_RB, DF), lambda i: (i, 0))],
      out_shape=[jax.ShapeDtypeStruct((N, DF), jnp.float32)],
  )(s0, s1, v0, v1, di, b20, b21, w30, w31, b30, b31, w40, w41, b40, b41)[0]


def _dot(a, b):
  def body(a_ref, b_ref, o_ref):
    o_ref[...] = jnp.sum(a_ref[...] * b_ref[...], axis=1, keepdims=True)
  return pl.pallas_call(
      body,
      grid=(LPAD // 2048,),
      in_specs=[
          pl.BlockSpec((2048, DF), lambda i: (i, 0)),
          pl.BlockSpec((2048, DF), lambda i: (i, 0)),
      ],
      out_specs=[pl.BlockSpec((2048, 1), lambda i: (i, 0))],
      out_shape=[jax.ShapeDtypeStruct((LPAD, 1), jnp.float32)],
  )(a, b)[0]


def kernel(x, edge_index, edge_label_index,
           p0_W1, p0_b1, p0_W2, p0_b2, p0_W3, p0_b3, p0_W4, p0_b4,
           p1_W1, p1_b1, p1_W2, p1_b2, p1_W3, p1_b3, p1_W4, p1_b4):
  pad_e = EPAD - E
  src = jnp.concatenate(
      [edge_index[0], jnp.zeros((pad_e,), jnp.int32)]).reshape(EPAD // 128, 128)
  dst_flat = jnp.concatenate(
      [edge_index[1], jnp.full((pad_e,), N, jnp.int32)])
  dst = dst_flat.reshape(EPAD // 128, 128)
  dump = DUMP + (jnp.arange(EPAD, dtype=jnp.int32) % 48)
  dsta = jnp.where(dst_flat < NH, dst_flat, dump).reshape(EPAD // 128, 128)
  dstb = jnp.where((dst_flat >= NH) & (dst_flat < 2 * NH), dst_flat - NH,
                   dump).reshape(EPAD // 128, 128)
  dstc = jnp.where((dst_flat >= 2 * NH) & (dst_flat < N), dst_flat - 2 * NH,
                   dump).reshape(EPAD // 128, 128)
  pad_l = LPAD - NL
  ls = jnp.concatenate([edge_label_index[0], jnp.zeros((pad_l,), jnp.int32)])
  ld = jnp.concatenate([edge_label_index[1], jnp.zeros((pad_l,), jnp.int32)])

  degp = _deg_kernel(dst)
  degcol = _deg_sum(degp).reshape(NPAD, 1)[:N]

  v0, v1, di = _dense1(x, p0_W1, p1_W1, degcol)
  s0a, s0b, s0c, s1a, s1b, s1c = _prop_kernel(v0, v1, src, dsta, dstb, dstc)
  s0 = jnp.concatenate([s0a[:NH], s0b[:NH], s0c[:N - 2 * NH]])
  s1 = jnp.concatenate([s1a[:NH], s1b[:NH], s1c[:N - 2 * NH]])
  v20, v21 = _dense2(s0, s1, v0, v1, di,
                     p0_W2, p1_W2,
                     p0_b1.reshape(1, DF), p1_b1.reshape(1, DF))
  t0a, t0b, t0c, t1a, t1b, t1c = _prop_kernel(v20, v21, src, dsta, dstb, dstc)
  t0 = jnp.concatenate([t0a[:NH], t0b[:NH], t0c[:N - 2 * NH]])
  t1 = jnp.concatenate([t1a[:NH], t1b[:NH], t1c[:N - 2 * NH]])
  z = _dense3(t0, t1, v20, v21, di,
              p0_b2.reshape(1, DF), p1_b2.reshape(1, DF),
              p0_W3, p1_W3, p0_b3.reshape(1, DF), p1_b3.reshape(1, DF),
              p0_W4, p1_W4, p0_b4.reshape(1, DF), p1_b4.reshape(1, DF))

  zs, zd = _decode_gather(z, ls, ld)
  out = _dot(zs, zd)
  return out[:NL, 0]


# speedup vs baseline: 1.0606x; 1.0606x over previous
import functools

import jax
import jax.numpy as jnp
from jax import lax
from jax.experimental import pallas as pl
from jax.experimental.pallas import tpu as pltpu
from jax.experimental.pallas import tpu_sc as plsc

N = 10000
NH = 3392
DF = 128
E = 320000
NL = 100000

NC = 2
NS = 16
APAD = 3456
DUMP = 3400
EPAD = 327680
ECH_T = 160
ECH_W = 80
NPAD = 10112
LPAD = 102400
LCH_W = 25

_MESH = plsc.VectorSubcoreMesh(
    core_axis_name="c", subcore_axis_name="s", num_cores=NC, num_subcores=NS)


def _fill16(ref, n, val):
  def body(i, _):
    ref[i, :] = jnp.full((16,), val, jnp.float32)
  lax.fori_loop(0, n, body, None)


def _zero_rows(zref, n):
  def body(i, _):
    for j in range(8):
      zref[i, pl.ds(j * 16, 16)] = jnp.zeros((16,), jnp.float32)
  lax.fori_loop(0, n, body, None)


@functools.partial(
    pl.kernel,
    out_type=jax.ShapeDtypeStruct((NC * NS, NPAD // 128, 128), jnp.float32),
    mesh=_MESH,
    scratch_types=[
        pltpu.VMEM((ECH_W, 128), jnp.int32),
        pltpu.VMEM((NPAD // 128, 128), jnp.float32),
    ],
)
def _deg_kernel(dst_hbm, degp_hbm, idx_v, hist_v):
  cid = lax.axis_index("c")
  sid = lax.axis_index("s")
  wid = sid * NC + cid
  _zero_rows(hist_v, NPAD // 128)
  pltpu.sync_copy(dst_hbm.at[pl.ds(wid * ECH_W, ECH_W)], idx_v)

  lanes = lax.iota(jnp.int32, 16)

  def upd(g, _):
    dvec = idx_v[g // 8, pl.ds((g % 8) * 16, 16)]
    for l in range(16):
      d = dvec[l]
      hr = d // 128
      hs = (d % 128) // 16 * 16
      hm = d % 16
      seg = hist_v[hr, pl.ds(hs, 16)]
      hist_v[hr, pl.ds(hs, 16)] = jnp.where(lanes == hm, seg + 1.0, seg)
  lax.fori_loop(0, ECH_W * 8, upd, None)
  pltpu.sync_copy(hist_v, degp_hbm.at[wid])


def _deg_sum(degp):
  def body(p_ref, o_ref):
    o_ref[...] = jnp.sum(p_ref[...], axis=0)
  return pl.pallas_call(
      body,
      grid=(1,),
      in_specs=[pl.BlockSpec((NC * NS, NPAD // 128, 128),
                             lambda i: (0, 0, 0))],
      out_specs=[pl.BlockSpec((NPAD // 128, 128), lambda i: (0, 0))],
      out_shape=[jax.ShapeDtypeStruct((NPAD // 128, 128), jnp.float32)],
  )(degp)[0]


@functools.partial(
    pl.kernel,
    out_type=[
        jax.ShapeDtypeStruct((APAD, DF), jnp.float32),
        jax.ShapeDtypeStruct((APAD, DF), jnp.float32),
        jax.ShapeDtypeStruct((APAD, DF), jnp.float32),
        jax.ShapeDtypeStruct((APAD, DF), jnp.float32),
        jax.ShapeDtypeStruct((APAD, DF), jnp.float32),
        jax.ShapeDtypeStruct((APAD, DF), jnp.float32),
    ],
    mesh=_MESH,
    scratch_types=[
        pltpu.VMEM((ECH_T, 128), jnp.int32),
        pltpu.VMEM((ECH_T, 128), jnp.int32),
        pltpu.VMEM((128, DF), jnp.float32),
        pltpu.VMEM((128, DF), jnp.float32),
        pltpu.VMEM((72, DF), jnp.float32),
        pltpu.VMEM_SHARED((APAD, DF), jnp.float32),
        pltpu.SemaphoreType.DMA,
        pltpu.SemaphoreType.DMA,
    ],
)
def _prop_kernel(v0_hbm, v1_hbm, src_hbm, dsta_hbm, dstb_hbm, dstc_hbm,
                 s0a_hbm, s0b_hbm, s0c_hbm, s1a_hbm, s1b_hbm, s1c_hbm,
                 srcv, dstv, rows_a, rows_b, zero_v, acc,
                 sem_a, sem_b):
  cid = lax.axis_index("c")
  sid = lax.axis_index("s")
  _zero_rows(zero_v, 72)

  pltpu.sync_copy(src_hbm.at[pl.ds(sid * ECH_T, ECH_T)], srcv)

  def one_pass(v_hbm, dst_hbm, s_hbm):
    pltpu.sync_copy(dst_hbm.at[pl.ds(sid * ECH_T, ECH_T)], dstv)
    for k in range(3):
      pltpu.sync_copy(zero_v, acc.at[pl.ds(sid * 216 + k * 72, 72)])
    plsc.subcore_barrier()

    pltpu.async_copy(v_hbm.at[srcv.at[0]], rows_a, sem_a)

    def step(g, _):
      ja = 2 * g
      jb = 2 * g + 1
      pltpu.make_async_copy(v_hbm.at[srcv.at[ja]], rows_a, sem_a).wait()
      pltpu.async_copy(v_hbm.at[srcv.at[jb]], rows_b, sem_b)
      pltpu.sync_copy(rows_a, acc.at[dstv.at[ja]], add=True)
      pltpu.make_async_copy(v_hbm.at[srcv.at[jb]], rows_b, sem_b).wait()

      @pl.when(ja + 2 < ECH_T)
      def _():
        pltpu.async_copy(v_hbm.at[srcv.at[ja + 2]], rows_a, sem_a)
      pltpu.sync_copy(rows_b, acc.at[dstv.at[jb]], add=True)
    lax.fori_loop(0, ECH_T // 2, step, None)
    plsc.subcore_barrier()
    pltpu.sync_copy(acc.at[pl.ds(sid * 216, 216)],
                    s_hbm.at[pl.ds(sid * 216, 216)])

  @pl.when(cid == 0)
  def _():
    one_pass(v0_hbm, dsta_hbm, s0a_hbm)
    one_pass(v0_hbm, dstb_hbm, s0b_hbm)
    one_pass(v0_hbm, dstc_hbm, s0c_hbm)

  @pl.when(cid == 1)
  def _():
    one_pass(v1_hbm, dsta_hbm, s1a_hbm)
    one_pass(v1_hbm, dstb_hbm, s1b_hbm)
    one_pass(v1_hbm, dstc_hbm, s1c_hbm)


@functools.partial(
    pl.kernel,
    out_type=[
        jax.ShapeDtypeStruct((LPAD, DF), jnp.float32),
        jax.ShapeDtypeStruct((LPAD, DF), jnp.float32),
    ],
    mesh=_MESH,
    scratch_types=[
        pltpu.VMEM((LCH_W * 128,), jnp.int32),
        pltpu.VMEM((LCH_W * 128,), jnp.int32),
        pltpu.VMEM((128, DF), jnp.float32),
        pltpu.VMEM((128, DF), jnp.float32),
        pltpu.SemaphoreType.DMA,
        pltpu.SemaphoreType.DMA,
    ],
)
def _decode_gather(z_hbm, ls_hbm, ld_hbm, zs_hbm, zd_hbm,
                   idxs, idxd, rows_a, rows_b, sem_a, sem_b):
  cid = lax.axis_index("c")
  sid = lax.axis_index("s")
  wid = sid * NC + cid
  nw = LCH_W * 128
  pltpu.sync_copy(ls_hbm.at[pl.ds(wid * nw, nw)], idxs)
  pltpu.sync_copy(ld_hbm.at[pl.ds(wid * nw, nw)], idxd)

  def step(j, _):
    isl = idxs.at[pl.ds(j * 128, 128)]
    idl = idxd.at[pl.ds(j * 128, 128)]
    base = wid * nw + j * 128
    pltpu.async_copy(z_hbm.at[isl], rows_a, sem_a)
    pltpu.async_copy(z_hbm.at[idl], rows_b, sem_b)
    pltpu.make_async_copy(z_hbm.at[isl], rows_a, sem_a).wait()
    pltpu.sync_copy(rows_a, zs_hbm.at[pl.ds(base, 128)])
    pltpu.make_async_copy(z_hbm.at[idl], rows_b, sem_b).wait()
    pltpu.sync_copy(rows_b, zd_hbm.at[pl.ds(base, 128)])
  lax.fori_loop(0, LCH_W, step, None)


_RB = 1000


def _dense1(x, w0, w1, degcol):
  def body(x_ref, w0_ref, w1_ref, dc_ref, v0_ref, v1_ref, di_ref):
    di = lax.rsqrt(dc_ref[...] + 1.0)
    xb = x_ref[...]
    v0_ref[...] = di * jnp.dot(xb, w0_ref[...], preferred_element_type=jnp.float32)
    v1_ref[...] = di * jnp.dot(xb, w1_ref[...], preferred_element_type=jnp.float32)
    di_ref[...] = di
  return pl.pallas_call(
      body,
      grid=(N // _RB,),
      in_specs=[
          pl.BlockSpec((_RB, DF), lambda i: (i, 0)),
          pl.BlockSpec((DF, DF), lambda i: (0, 0)),
          pl.BlockSpec((DF, DF), lambda i: (0, 0)),
          pl.BlockSpec((_RB, 1), lambda i: (i, 0)),
      ],
      out_specs=[
          pl.BlockSpec((_RB, DF), lambda i: (i, 0)),
          pl.BlockSpec((_RB, DF), lambda i: (i, 0)),
          pl.BlockSpec((_RB, 1), lambda i: (i, 0)),
      ],
      out_shape=[
          jax.ShapeDtypeStruct((N, DF), jnp.float32),
          jax.ShapeDtypeStruct((N, DF), jnp.float32),
          jax.ShapeDtypeStruct((N, 1), jnp.float32),
      ],
  )(x, w0, w1, degcol)


def _dense2(s0, s1, v0, v1, di, w20, w21, b10, b11):
  def body(s0_ref, s1_ref, v0_ref, v1_ref, di_ref, w20_ref, w21_ref,
           b10_ref, b11_ref, o0_ref, o1_ref):
    d = di_ref[...]
    h0 = d * (s0_ref[...] + v0_ref[...]) + b10_ref[...]
    h1 = d * (s1_ref[...] + v1_ref[...]) + b11_ref[...]
    o0_ref[...] = d * jnp.dot(h0, w20_ref[...], preferred_element_type=jnp.float32)
    o1_ref[...] = d * jnp.dot(h1, w21_ref[...], preferred_element_type=jnp.float32)
  return pl.pallas_call(
      body,
      grid=(N // _RB,),
      in_specs=[
          pl.BlockSpec((_RB, DF), lambda i: (i, 0)),
          pl.BlockSpec((_RB, DF), lambda i: (i, 0)),
          pl.BlockSpec((_RB, DF), lambda i: (i, 0)),
          pl.BlockSpec((_RB, DF), lambda i: (i, 0)),
          pl.BlockSpec((_RB, 1), lambda i: (i, 0)),
          pl.BlockSpec((DF, DF), lambda i: (0, 0)),
          pl.BlockSpec((DF, DF), lambda i: (0, 0)),
          pl.BlockSpec((1, DF), lambda i: (0, 0)),
          pl.BlockSpec((1, DF), lambda i: (0, 0)),
      ],
      out_specs=[
          pl.BlockSpec((_RB, DF), lambda i: (i, 0)),
          pl.BlockSpec((_RB, DF), lambda i: (i, 0)),
      ],
      out_shape=[
          jax.ShapeDtypeStruct((N, DF), jnp.float32),
          jax.ShapeDtypeStruct((N, DF), jnp.float32),
      ],
  )(s0, s1, v0, v1, di, w20, w21, b10, b11)


def _dense3(s0, s1, v0, v1, di, b20, b21, w30, w31, b30, b31,
            w40, w41, b40, b41):
  def body(s0_ref, s1_ref, v0_ref, v1_ref, di_ref, b20_ref, b21_ref,
           w30_ref, w31_ref, b30_ref, b31_ref, w40_ref, w41_ref,
           b40_ref, b41_ref, z_ref):
    d = di_ref[...]
    h0 = d * (s0_ref[...] + v0_ref[...]) + b20_ref[...]
    h1 = d * (s1_ref[...] + v1_ref[...]) + b21_ref[...]
    g0 = jnp.maximum(
        jnp.dot(h0, w30_ref[...], preferred_element_type=jnp.float32)
        + b30_ref[...], 0.0)
    g1 = jnp.maximum(
        jnp.dot(h1, w31_ref[...], preferred_element_type=jnp.float32)
        + b31_ref[...], 0.0)
    z0 = jnp.dot(g0, w40_ref[...], preferred_element_type=jnp.float32) + b40_ref[...]
    z1 = jnp.dot(g1, w41_ref[...], preferred_element_type=jnp.float32) + b41_ref[...]
    z_ref[...] = 0.5 * (z0 + z1)
  full = lambda i: (0, 0)
  return pl.pallas_call(
      body,
      grid=(N // _RB,),
      in_specs=[
          pl.BlockSpec((_RB, DF), lambda i: (i, 0)),
          pl.BlockSpec((_RB, DF), lambda i: (i, 0)),
          pl.BlockSpec((_RB, DF), lambda i: (i, 0)),
          pl.BlockSpec((_RB, DF), lambda i: (i, 0)),
          pl.BlockSpec((_RB, 1), lambda i: (i, 0)),
          pl.BlockSpec((1, DF), full),
          pl.BlockSpec((1, DF), full),
          pl.BlockSpec((DF, DF), full),
          pl.BlockSpec((DF, DF), full),
          pl.BlockSpec((1, DF), full),
          pl.BlockSpec((1, DF), full),
          pl.BlockSpec((DF, DF), full),
          pl.BlockSpec((DF, DF), full),
          pl.BlockSpec((1, DF), full),
          pl.BlockSpec((1, DF), full),
      ],
      out_specs=[pl.BlockSpec((_RB, DF), lambda i: (i, 0))],
      out_shape=[jax.ShapeDtypeStruct((N, DF), jnp.float32)],
  )(s0, s1, v0, v1, di, b20, b21, w30, w31, b30, b31, w40, w41, b40, b41)[0]


def _dot(a, b):
  def body(a_ref, b_ref, o_ref):
    o_ref[...] = jnp.sum(a_ref[...] * b_ref[...], axis=1, keepdims=True)
  return pl.pallas_call(
      body,
      grid=(LPAD // 2048,),
      in_specs=[
          pl.BlockSpec((2048, DF), lambda i: (i, 0)),
          pl.BlockSpec((2048, DF), lambda i: (i, 0)),
      ],
      out_specs=[pl.BlockSpec((2048, 1), lambda i: (i, 0))],
      out_shape=[jax.ShapeDtypeStruct((LPAD, 1), jnp.float32)],
  )(a, b)[0]


def kernel(x, edge_index, edge_label_index,
           p0_W1, p0_b1, p0_W2, p0_b2, p0_W3, p0_b3, p0_W4, p0_b4,
           p1_W1, p1_b1, p1_W2, p1_b2, p1_W3, p1_b3, p1_W4, p1_b4):
  pad_e = EPAD - E
  src = jnp.concatenate(
      [edge_index[0], jnp.zeros((pad_e,), jnp.int32)]).reshape(EPAD // 128, 128)
  dst_flat = jnp.concatenate(
      [edge_index[1], jnp.full((pad_e,), N, jnp.int32)])
  dst = dst_flat.reshape(EPAD // 128, 128)
  dsta = jnp.where(dst_flat < NH, dst_flat, DUMP).reshape(EPAD // 128, 128)
  dstb = jnp.where((dst_flat >= NH) & (dst_flat < 2 * NH), dst_flat - NH,
                   DUMP).reshape(EPAD // 128, 128)
  dstc = jnp.where((dst_flat >= 2 * NH) & (dst_flat < N), dst_flat - 2 * NH,
                   DUMP).reshape(EPAD // 128, 128)
  pad_l = LPAD - NL
  ls = jnp.concatenate([edge_label_index[0], jnp.zeros((pad_l,), jnp.int32)])
  ld = jnp.concatenate([edge_label_index[1], jnp.zeros((pad_l,), jnp.int32)])

  degp = _deg_kernel(dst)
  degcol = _deg_sum(degp).reshape(NPAD, 1)[:N]

  v0, v1, di = _dense1(x, p0_W1, p1_W1, degcol)
  s0a, s0b, s0c, s1a, s1b, s1c = _prop_kernel(v0, v1, src, dsta, dstb, dstc)
  s0 = jnp.concatenate([s0a[:NH], s0b[:NH], s0c[:N - 2 * NH]])
  s1 = jnp.concatenate([s1a[:NH], s1b[:NH], s1c[:N - 2 * NH]])
  v20, v21 = _dense2(s0, s1, v0, v1, di,
                     p0_W2, p1_W2,
                     p0_b1.reshape(1, DF), p1_b1.reshape(1, DF))
  t0a, t0b, t0c, t1a, t1b, t1c = _prop_kernel(v20, v21, src, dsta, dstb, dstc)
  t0 = jnp.concatenate([t0a[:NH], t0b[:NH], t0c[:N - 2 * NH]])
  t1 = jnp.concatenate([t1a[:NH], t1b[:NH], t1c[:N - 2 * NH]])
  z = _dense3(t0, t1, v20, v21, di,
              p0_b2.reshape(1, DF), p1_b2.reshape(1, DF),
              p0_W3, p1_W3, p0_b3.reshape(1, DF), p1_b3.reshape(1, DF),
              p0_W4, p1_W4, p0_b4.reshape(1, DF), p1_b4.reshape(1, DF))

  zs, zd = _decode_gather(z, ls, ld)
  out = _dot(zs, zd)
  return out[:NL, 0]
